# baseline (speedup 1.0000x reference)
import functools

import jax
import jax.numpy as jnp
from jax import lax
from jax.experimental import pallas as pl
from jax.experimental.pallas import tpu as pltpu, tpu_sc as plsc

_S = 1000000
_A = 64
_B = 16384
_NC, _NS, _L = 2, 16, 16
_NW = _NC * _NS
_BPW = _B // _NW
_NGRP = _BPW // _L
_CB = 65536
_NBLK = (_S + _CB - 1) // _CB
_MROWS = _NBLK * (_CB // 128)
_CHUNK = 128


def _tc_colmax(qt_ref, out_ref):
    m = jnp.max(qt_ref[...], axis=0).reshape(_CB // 128, 128)

    @pl.when(pl.program_id(1) == 0)
    def _init():
        out_ref[...] = m

    @pl.when(pl.program_id(1) > 0)
    def _acc():
        out_ref[...] = jnp.maximum(out_ref[...], m)


_DEPTH = 6


def _sc_qsa(qt, st, ac, qsa_out, st_v, ac_v, qsa_v, ring,
            sem0, sem1, sem2, sem3, sem4, sem5):
    wid = lax.axis_index("s") * _NC + lax.axis_index("c")
    base = wid * _BPW
    pltpu.sync_copy(st.at[pl.ds(base, _BPW)], st_v)
    pltpu.sync_copy(ac.at[pl.ds(base, _BPW)], ac_v)

    sems = (sem0, sem1, sem2, sem3, sem4, sem5)
    iota = lax.iota(jnp.int32, _L)

    def fire(g):
        p = g % _DEPTH
        sl = pl.ds(g * _L, _L)
        a16 = ac_v[sl]
        s16 = st_v[sl]
        for k in range(_L):
            ab = pl.multiple_of((a16[k] >> 3) << 3, 8)
            sb = pl.multiple_of((s16[k] >> 7) << 7, 128)
            pltpu.async_copy(qt.at[pl.ds(ab, 8), pl.ds(sb, 128)],
                             ring.at[p * _L + k], sems[p])

    def drain(g):
        p = g % _DEPTH

        def go(i, carry):
            pltpu.make_async_copy(qt.at[pl.ds(0, 8), pl.ds(0, 128)],
                                  ring.at[p * _L + i], sems[p]).wait()
            return carry
        lax.fori_loop(0, _L, go, 0)

    for g in range(_DEPTH - 1):
        fire(g)
    for g in range(_NGRP):
        if g + _DEPTH - 1 < _NGRP:
            fire(g + _DEPTH - 1)
        drain(g)
        p = g % _DEPTH
        sl = pl.ds(g * _L, _L)
        qsa_v[sl] = plsc.load_gather(
            ring, [p * _L + iota, ac_v[sl] & 7, st_v[sl] & 127])

    pltpu.sync_copy(qsa_v, qsa_out.at[pl.ds(base, _BPW)])


def _sc_finish(qmax, qsa, rw, ic, ns, out,
               ns_v, qsa_v, rw_v, ic_v, idxm, qrows, out_v, semg):
    wid = lax.axis_index("s") * _NC + lax.axis_index("c")
    base = wid * _BPW
    pltpu.sync_copy(ns.at[pl.ds(base, _BPW)], ns_v)
    pltpu.sync_copy(qsa.at[pl.ds(base, _BPW)], qsa_v)
    pltpu.sync_copy(rw.at[pl.ds(base, _BPW)], rw_v)
    pltpu.sync_copy(ic.at[pl.ds(base, _BPW)], ic_v)

    def build(j, carry):
        sl = pl.ds(j * _L, _L)
        idxm[sl] = ns_v[sl] >> 7
        return carry
    lax.fori_loop(0, _NGRP, build, 0)

    copies = []
    for k in range(_BPW // _CHUNK):
        cs = pl.ds(k * _CHUNK, _CHUNK)
        copies.append(pltpu.async_copy(qmax.at[idxm.at[cs]], qrows.at[cs], semg))
    for c in copies:
        c.wait()

    iota = lax.iota(jnp.int32, _L)

    def fin(j, carry):
        sl = pl.ds(j * _L, _L)
        m = plsc.load_gather(qrows, [j * _L + iota, ns_v[sl] & 127])
        out_v[sl] = jnp.abs(rw_v[sl] + m - qsa_v[sl]) * ic_v[sl]
        return carry
    lax.fori_loop(0, _NGRP, fin, 0)

    pltpu.sync_copy(out_v, out.at[pl.ds(base, _BPW)])


def kernel(q_table, reward, instruction_context, state, action, next_state):
    qt = q_table.T
    st = state.astype(jnp.int32)
    ac = action.astype(jnp.int32)
    ns = next_state.astype(jnp.int32)

    qsa = pl.kernel(
        _sc_qsa,
        out_type=jax.ShapeDtypeStruct((_B,), jnp.float32),
        mesh=plsc.VectorSubcoreMesh(core_axis_name="c", subcore_axis_name="s"),
        scratch_types=[
            pltpu.VMEM((_BPW,), jnp.int32),
            pltpu.VMEM((_BPW,), jnp.int32),
            pltpu.VMEM((_BPW,), jnp.float32),
            pltpu.VMEM((_DEPTH * _L, 8, 128), jnp.float32),
            pltpu.SemaphoreType.DMA,
            pltpu.SemaphoreType.DMA,
            pltpu.SemaphoreType.DMA,
            pltpu.SemaphoreType.DMA,
            pltpu.SemaphoreType.DMA,
            pltpu.SemaphoreType.DMA,
        ],
        compiler_params=pltpu.CompilerParams(needs_layout_passes=False),
        cost_estimate=pl.CostEstimate(
            flops=2 * _B, transcendentals=0, bytes_accessed=_B * 4096),
    )(qt, st, ac)

    qmax = pl.pallas_call(
        _tc_colmax,
        grid=(_NBLK, _A // 8),
        in_specs=[pl.BlockSpec((8, _CB), lambda i, j: (j, i))],
        out_specs=pl.BlockSpec((_CB // 128, 128), lambda i, j: (i, 0)),
        out_shape=jax.ShapeDtypeStruct((_MROWS, 128), jnp.float32),
    )(qt)

    out = pl.kernel(
        _sc_finish,
        out_type=jax.ShapeDtypeStruct((_B,), jnp.float32),
        mesh=plsc.VectorSubcoreMesh(core_axis_name="c", subcore_axis_name="s"),
        scratch_types=[
            pltpu.VMEM((_BPW,), jnp.int32),
            pltpu.VMEM((_BPW,), jnp.float32),
            pltpu.VMEM((_BPW,), jnp.float32),
            pltpu.VMEM((_BPW,), jnp.float32),
            pltpu.VMEM((_BPW,), jnp.int32),
            pltpu.VMEM((_BPW, 128), jnp.float32),
            pltpu.VMEM((_BPW,), jnp.float32),
            pltpu.SemaphoreType.DMA,
        ],
        compiler_params=pltpu.CompilerParams(needs_layout_passes=False),
    )(qmax, qsa, reward, instruction_context, ns)

    return out


# speedup vs baseline: 1.4242x; 1.4242x over previous
import functools

import jax
import jax.numpy as jnp
from jax import lax
from jax.experimental import pallas as pl
from jax.experimental.pallas import tpu as pltpu, tpu_sc as plsc

_S = 1000000
_A = 64
_B = 16384
_NC, _NS, _L = 2, 16, 16
_NW = _NC * _NS
_BPW = _B // _NW
_NGRP = _BPW // _L
_CB = 32768
_NBLK = (_S + _CB - 1) // _CB
_MROWS = _NBLK * (_CB // 128)
_CHUNK = 128


def _tc_colmax(qt_ref, out_ref):
    out_ref[...] = jnp.max(qt_ref[...], axis=0).reshape(_CB // 128, 128)


_DEPTH = 6


def _sc_qsa(qt, st, ac, qsa_out, st_v, ac_v, qsa_v, ring,
            sem0, sem1, sem2, sem3, sem4, sem5):
    wid = lax.axis_index("s") * _NC + lax.axis_index("c")
    base = wid * _BPW
    pltpu.sync_copy(st.at[pl.ds(base, _BPW)], st_v)
    pltpu.sync_copy(ac.at[pl.ds(base, _BPW)], ac_v)

    sems = (sem0, sem1, sem2, sem3, sem4, sem5)
    iota = lax.iota(jnp.int32, _L)

    def fire(g):
        p = g % _DEPTH
        sl = pl.ds(g * _L, _L)
        a16 = ac_v[sl]
        s16 = st_v[sl]
        for k in range(_L):
            ab = pl.multiple_of((a16[k] >> 3) << 3, 8)
            sb = pl.multiple_of((s16[k] >> 7) << 7, 128)
            pltpu.async_copy(qt.at[pl.ds(ab, 8), pl.ds(sb, 128)],
                             ring.at[p * _L + k], sems[p])

    def drain(g):
        p = g % _DEPTH

        def go(i, carry):
            pltpu.make_async_copy(qt.at[pl.ds(0, 8), pl.ds(0, 128)],
                                  ring.at[p * _L + i], sems[p]).wait()
            return carry
        lax.fori_loop(0, _L, go, 0)

    for g in range(_DEPTH - 1):
        fire(g)
    for g in range(_NGRP):
        if g + _DEPTH - 1 < _NGRP:
            fire(g + _DEPTH - 1)
        drain(g)
        p = g % _DEPTH
        sl = pl.ds(g * _L, _L)
        qsa_v[sl] = plsc.load_gather(
            ring, [p * _L + iota, ac_v[sl] & 7, st_v[sl] & 127])

    pltpu.sync_copy(qsa_v, qsa_out.at[pl.ds(base, _BPW)])


def _sc_finish(qmax, qsa, rw, ic, ns, out,
               ns_v, qsa_v, rw_v, ic_v, idxm, qrows, out_v, semg):
    wid = lax.axis_index("s") * _NC + lax.axis_index("c")
    base = wid * _BPW
    pltpu.sync_copy(ns.at[pl.ds(base, _BPW)], ns_v)
    pltpu.sync_copy(qsa.at[pl.ds(base, _BPW)], qsa_v)
    pltpu.sync_copy(rw.at[pl.ds(base, _BPW)], rw_v)
    pltpu.sync_copy(ic.at[pl.ds(base, _BPW)], ic_v)

    def build(j, carry):
        sl = pl.ds(j * _L, _L)
        idxm[sl] = ns_v[sl] >> 7
        return carry
    lax.fori_loop(0, _NGRP, build, 0)

    copies = []
    for k in range(_BPW // _CHUNK):
        cs = pl.ds(k * _CHUNK, _CHUNK)
        copies.append(pltpu.async_copy(qmax.at[idxm.at[cs]], qrows.at[cs], semg))
    for c in copies:
        c.wait()

    iota = lax.iota(jnp.int32, _L)

    def fin(j, carry):
        sl = pl.ds(j * _L, _L)
        m = plsc.load_gather(qrows, [j * _L + iota, ns_v[sl] & 127])
        out_v[sl] = jnp.abs(rw_v[sl] + m - qsa_v[sl]) * ic_v[sl]
        return carry
    lax.fori_loop(0, _NGRP, fin, 0)

    pltpu.sync_copy(out_v, out.at[pl.ds(base, _BPW)])


def kernel(q_table, reward, instruction_context, state, action, next_state):
    qt = q_table.T
    st = state.astype(jnp.int32)
    ac = action.astype(jnp.int32)
    ns = next_state.astype(jnp.int32)

    qsa = pl.kernel(
        _sc_qsa,
        out_type=jax.ShapeDtypeStruct((_B,), jnp.float32),
        mesh=plsc.VectorSubcoreMesh(core_axis_name="c", subcore_axis_name="s"),
        scratch_types=[
            pltpu.VMEM((_BPW,), jnp.int32),
            pltpu.VMEM((_BPW,), jnp.int32),
            pltpu.VMEM((_BPW,), jnp.float32),
            pltpu.VMEM((_DEPTH * _L, 8, 128), jnp.float32),
            pltpu.SemaphoreType.DMA,
            pltpu.SemaphoreType.DMA,
            pltpu.SemaphoreType.DMA,
            pltpu.SemaphoreType.DMA,
            pltpu.SemaphoreType.DMA,
            pltpu.SemaphoreType.DMA,
        ],
        compiler_params=pltpu.CompilerParams(needs_layout_passes=False),
        cost_estimate=pl.CostEstimate(
            flops=2 * _B, transcendentals=0, bytes_accessed=_B * 4096),
    )(qt, st, ac)

    qmax = pl.pallas_call(
        _tc_colmax,
        grid=(_NBLK,),
        in_specs=[pl.BlockSpec((_A, _CB), lambda i: (0, i))],
        out_specs=pl.BlockSpec((_CB // 128, 128), lambda i: (i, 0)),
        out_shape=jax.ShapeDtypeStruct((_MROWS, 128), jnp.float32),
    )(qt)

    out = pl.kernel(
        _sc_finish,
        out_type=jax.ShapeDtypeStruct((_B,), jnp.float32),
        mesh=plsc.VectorSubcoreMesh(core_axis_name="c", subcore_axis_name="s"),
        scratch_types=[
            pltpu.VMEM((_BPW,), jnp.int32),
            pltpu.VMEM((_BPW,), jnp.float32),
            pltpu.VMEM((_BPW,), jnp.float32),
            pltpu.VMEM((_BPW,), jnp.float32),
            pltpu.VMEM((_BPW,), jnp.int32),
            pltpu.VMEM((_BPW, 128), jnp.float32),
            pltpu.VMEM((_BPW,), jnp.float32),
            pltpu.SemaphoreType.DMA,
        ],
        compiler_params=pltpu.CompilerParams(needs_layout_passes=False),
    )(qmax, qsa, reward, instruction_context, ns)

    return out


# speedup vs baseline: 1.4287x; 1.0032x over previous
import jax
import jax.numpy as jnp
from jax import lax
from jax.experimental import pallas as pl
from jax.experimental.pallas import tpu as pltpu, tpu_sc as plsc

_S = 1000000
_A = 64
_B = 16384
_NC, _NS, _L = 2, 16, 16
_NW = _NC * _NS
_BPW = _B // _NW
_NGRP = _BPW // _L
_CB = 32768
_NBLK = (_S + _CB - 1) // _CB
_MROWS = _NBLK * (_CB // 128)
_CHUNK = 128


def _tc_colmax(qt_ref, out_ref):
    out_ref[...] = jnp.max(qt_ref[...], axis=0).reshape(_CB // 128, 128)


_DEPTH = 6


def _sc_qsa(qt, st, ac, qsa_out, st_v, ac_v, qsa_v, ring,
            sem0, sem1, sem2, sem3, sem4, sem5):
    wid = lax.axis_index("s") * _NC + lax.axis_index("c")
    base = wid * _BPW
    pltpu.sync_copy(st.at[pl.ds(base, _BPW)], st_v)
    pltpu.sync_copy(ac.at[pl.ds(base, _BPW)], ac_v)

    sems = (sem0, sem1, sem2, sem3, sem4, sem5)
    iota = lax.iota(jnp.int32, _L)

    def fire(g):
        p = g % _DEPTH
        sl = pl.ds(g * _L, _L)
        a16 = ac_v[sl]
        s16 = st_v[sl]
        for k in range(_L):
            ab = pl.multiple_of((a16[k] >> 3) << 3, 8)
            sb = pl.multiple_of((s16[k] >> 7) << 7, 128)
            pltpu.async_copy(qt.at[pl.ds(ab, 8), pl.ds(sb, 128)],
                             ring.at[p * _L + k], sems[p])

    def drain(g):
        p = g % _DEPTH

        def go(i, carry):
            pltpu.make_async_copy(qt.at[pl.ds(0, 8), pl.ds(0, 128)],
                                  ring.at[p * _L + i], sems[p]).wait()
            return carry
        lax.fori_loop(0, _L, go, 0)

    for g in range(_DEPTH - 1):
        fire(g)
    for g in range(_NGRP):
        if g + _DEPTH - 1 < _NGRP:
            fire(g + _DEPTH - 1)
        drain(g)
        p = g % _DEPTH
        sl = pl.ds(g * _L, _L)
        qsa_v[sl] = plsc.load_gather(
            ring, [p * _L + iota, ac_v[sl] & 7, st_v[sl] & 127])

    pltpu.sync_copy(qsa_v, qsa_out.at[pl.ds(base, _BPW)])


def _sc_finish(qmax, qsa, rw, ic, ns, out,
               ns_v, qsa_v, rw_v, ic_v, idxm, qrows, out_v, semg):
    wid = lax.axis_index("s") * _NC + lax.axis_index("c")
    base = wid * _BPW
    pltpu.sync_copy(ns.at[pl.ds(base, _BPW)], ns_v)
    pltpu.sync_copy(qsa.at[pl.ds(base, _BPW)], qsa_v)
    pltpu.sync_copy(rw.at[pl.ds(base, _BPW)], rw_v)
    pltpu.sync_copy(ic.at[pl.ds(base, _BPW)], ic_v)

    def build(j, carry):
        sl = pl.ds(j * _L, _L)
        idxm[sl] = ns_v[sl] >> 7
        return carry
    lax.fori_loop(0, _NGRP, build, 0)

    copies = []
    for k in range(_BPW // _CHUNK):
        cs = pl.ds(k * _CHUNK, _CHUNK)
        copies.append(pltpu.async_copy(qmax.at[idxm.at[cs]], qrows.at[cs], semg))
    for c in copies:
        c.wait()

    iota = lax.iota(jnp.int32, _L)

    def fin(j, carry):
        sl = pl.ds(j * _L, _L)
        m = plsc.load_gather(qrows, [j * _L + iota, ns_v[sl] & 127])
        out_v[sl] = jnp.abs(rw_v[sl] + m - qsa_v[sl]) * ic_v[sl]
        return carry
    lax.fori_loop(0, _NGRP, fin, 0)

    pltpu.sync_copy(out_v, out.at[pl.ds(base, _BPW)])


def kernel(q_table, reward, instruction_context, state, action, next_state):
    qt = q_table.T
    st = state.astype(jnp.int32)
    ac = action.astype(jnp.int32)
    ns = next_state.astype(jnp.int32)

    qsa = pl.kernel(
        _sc_qsa,
        out_type=jax.ShapeDtypeStruct((_B,), jnp.float32),
        mesh=plsc.VectorSubcoreMesh(core_axis_name="c", subcore_axis_name="s"),
        scratch_types=[
            pltpu.VMEM((_BPW,), jnp.int32),
            pltpu.VMEM((_BPW,), jnp.int32),
            pltpu.VMEM((_BPW,), jnp.float32),
            pltpu.VMEM((_DEPTH * _L, 8, 128), jnp.float32),
            pltpu.SemaphoreType.DMA,
            pltpu.SemaphoreType.DMA,
            pltpu.SemaphoreType.DMA,
            pltpu.SemaphoreType.DMA,
            pltpu.SemaphoreType.DMA,
            pltpu.SemaphoreType.DMA,
        ],
        compiler_params=pltpu.CompilerParams(needs_layout_passes=False),
        cost_estimate=pl.CostEstimate(
            flops=2 * _B, transcendentals=0, bytes_accessed=_B * 4096),
    )(qt, st, ac)

    qmax = pl.pallas_call(
        _tc_colmax,
        grid=(_NBLK,),
        in_specs=[pl.BlockSpec((_A, _CB), lambda i: (0, i))],
        out_specs=pl.BlockSpec((_CB // 128, 128), lambda i: (i, 0)),
        out_shape=jax.ShapeDtypeStruct((_MROWS, 128), jnp.float32),
    )(qt)

    out = pl.kernel(
        _sc_finish,
        out_type=jax.ShapeDtypeStruct((_B,), jnp.float32),
        mesh=plsc.VectorSubcoreMesh(core_axis_name="c", subcore_axis_name="s"),
        scratch_types=[
            pltpu.VMEM((_BPW,), jnp.int32),
            pltpu.VMEM((_BPW,), jnp.float32),
            pltpu.VMEM((_BPW,), jnp.float32),
            pltpu.VMEM((_BPW,), jnp.float32),
            pltpu.VMEM((_BPW,), jnp.int32),
            pltpu.VMEM((_BPW, 128), jnp.float32),
            pltpu.VMEM((_BPW,), jnp.float32),
            pltpu.SemaphoreType.DMA,
        ],
        compiler_params=pltpu.CompilerParams(needs_layout_passes=False),
    )(qmax, qsa, reward, instruction_context, ns)

    return out
